# baseline (speedup 1.0000x reference)
import jax
import jax.numpy as jnp
from jax import lax
from jax.experimental import pallas as pl
from jax.experimental.pallas import tpu as pltpu
from jax.experimental.pallas import tpu_sc as plsc

N = 10000
E = 640000
NC = 2
NS = 16
B0 = 4960
RNG = 80
EPT = E // NS
BLK = 1600
ABLK = 4800
GRP = 128
CAP = 43264

i32 = jnp.int32
f32 = jnp.float32


def _sc_mesh():
    return plsc.VectorSubcoreMesh(core_axis_name="c", subcore_axis_name="s")


def _make_partition():
    def body(key_arr, oth_arr, gsrc, gdst, cnts, key_st, oth_st, csrc, cdst, cnt_st):
        c = lax.axis_index("c")
        s = lax.axis_index("s")
        base_e = s * EPT
        half_hi = B0 + c * (2 * N)
        for bkt in range(2):
            lo = c * B0 + bkt * (32 * RNG)
            hi = jnp.minimum(lo + 32 * RNG, half_hi)

            def blk_body(b, cnt):
                off = base_e + b * BLK
                pltpu.sync_copy(key_arr.at[pl.ds(off, BLK)], key_st)
                pltpu.sync_copy(oth_arr.at[pl.ds(off, BLK)], oth_st)

                def in_body(i, cnt):
                    d = key_st[pl.ds(i * 16, 16)]
                    o = oth_st[pl.ds(i * 16, 16)]
                    m = (d >= lo) & (d < hi)
                    run = plsc.cumsum(m.astype(i32))
                    pos = cnt + run - 1
                    plsc.store_scatter(csrc, [pos], o, mask=m)
                    plsc.store_scatter(cdst, [pos], d, mask=m)
                    return cnt + run[15]

                return lax.fori_loop(0, BLK // 16, in_body, cnt)

            cnt = lax.fori_loop(0, EPT // BLK, blk_body, jnp.int32(0))
            slot = (c * 2 + bkt) * NS + s
            fbase = slot * CAP
            pltpu.sync_copy(csrc.at[pl.ds(0, CAP)], gsrc.at[pl.ds(fbase, CAP)])
            pltpu.sync_copy(cdst.at[pl.ds(0, CAP)], gdst.at[pl.ds(fbase, CAP)])
            cnt_st[...] = jnp.full((16,), cnt, i32)
            pltpu.sync_copy(cnt_st, cnts.at[pl.ds(slot * 16, 16)])

    return pl.kernel(
        body,
        out_type=[
            jax.ShapeDtypeStruct((NC * 2 * NS * CAP,), i32),
            jax.ShapeDtypeStruct((NC * 2 * NS * CAP,), i32),
            jax.ShapeDtypeStruct((NC * 2 * NS * 16,), i32),
        ],
        mesh=_sc_mesh(),
        compiler_params=pltpu.CompilerParams(needs_layout_passes=False),
        scratch_types=[
            pltpu.VMEM((BLK,), i32),
            pltpu.VMEM((BLK,), i32),
            pltpu.VMEM((CAP + GRP,), i32),
            pltpu.VMEM((CAP + GRP,), i32),
            pltpu.VMEM((16,), i32),
        ],
    )


def _make_agg(F: int):
    grp = 128 if F <= 256 else 64
    nj = 4 if F <= 256 else 2
    nscan = 1 if F <= 256 else 2
    nacc = nj * RNG

    def body(h, gsrc, gdst, cnts, out, sst, dst_, psrc, pldst, rows, acc, cnt_v, sem):
        c = lax.axis_index("c")
        s = lax.axis_index("s")
        nr = 62 + c
        half = c * B0
        pltpu.sync_copy(cnts, cnt_v)

        def fire():
            pltpu.async_copy(h.at[psrc.at[pl.ds(0, grp)]], rows, sem).wait()
            for g16 in range(grp // 16):
                ldv = pldst[pl.ds(g16 * 16, 16)]
                ds = [ldv[l] for l in range(16)]

                def addf(k, _, g16=g16, ds=ds):
                    for l in range(16):
                        v = rows[g16 * 16 + l, pl.ds(k * 16, 16)]
                        plsc.addupdate(acc.at[ds[l], pl.ds(k * 16, 16)], v)
                    return 0

                lax.fori_loop(0, F // 16, addf, 0)

        for q in range(nscan):
            def zrow(zr, _):
                def zb(k, _):
                    acc[zr, pl.ds(k * 16, 16)] = jnp.zeros((16,), f32)
                    return 0

                return lax.fori_loop(0, F // 16, zb, 0)

            lax.fori_loop(0, nacc + 8, zrow, 0)
            ridx = [s + 16 * (nj * q + j) for j in range(nj)]
            lo = [half + r * RNG for r in ridx]
            valid = [r < nr for r in ridx]
            slot0 = (c * 2 + (q if nscan == 2 else 0)) * NS
            nslot = NS if nscan == 2 else 2 * NS

            def scan_shard(sh, pend):
                slot = slot0 + sh
                cv = cnt_v[pl.ds(slot * 16, 16)]
                cnt_sh = cv[0]
                nblk = (cnt_sh + (ABLK - 1)) // ABLK

                def blkb(b, pend):
                    sbase = slot * CAP + b * ABLK
                    pltpu.sync_copy(gsrc.at[pl.ds(sbase, ABLK)], sst)
                    pltpu.sync_copy(gdst.at[pl.ds(sbase, ABLK)], dst_)

                    def inb(i, pend):
                        pos = b * ABLK + i * 16 + lax.broadcasted_iota(i32, (16,), 0)
                        d = dst_[pl.ds(i * 16, 16)]
                        o = sst[pl.ds(i * 16, 16)]
                        m = jnp.zeros((16,), jnp.bool_)
                        ld = jnp.full((16,), nacc, i32)
                        for j in range(nj):
                            inj = (d >= lo[j]) & (d < lo[j] + RNG) & valid[j]
                            m = m | inj
                            ld = jnp.where(inj, d - lo[j] + j * RNG, ld)
                        m = m & (pos < cnt_sh)
                        run = plsc.cumsum(m.astype(i32))
                        p = pend + run - 1
                        plsc.store_scatter(psrc, [p], o, mask=m)
                        plsc.store_scatter(pldst, [p], ld, mask=m)
                        pend = pend + run[15]

                        def do_fire(pend):
                            fire()
                            psrc[pl.ds(0, 16)] = psrc[pl.ds(grp, 16)]
                            pldst[pl.ds(0, 16)] = pldst[pl.ds(grp, 16)]
                            return pend - grp

                        return lax.cond(pend >= grp, do_fire, lambda p: p, pend)

                    return lax.fori_loop(0, ABLK // 16, inb, pend)

                return lax.fori_loop(0, nblk, blkb, pend)

            pend = lax.fori_loop(0, nslot, scan_shard, jnp.int32(0))
            for j in range(grp // 16):
                psrc[pl.ds(pend + j * 16, 16)] = jnp.zeros((16,), i32)
                pldst[pl.ds(pend + j * 16, 16)] = jnp.full((16,), nacc, i32)
            fire()
            for j in range(nj):
                @pl.when(valid[j])
                def _(j=j):
                    pltpu.sync_copy(
                        acc.at[pl.ds(j * RNG, RNG)],
                        out.at[pl.ds(lo[j], RNG)],
                    )

    return pl.kernel(
        body,
        out_type=jax.ShapeDtypeStruct((N, F), f32),
        mesh=_sc_mesh(),
        compiler_params=pltpu.CompilerParams(needs_layout_passes=False),
        scratch_types=[
            pltpu.VMEM((ABLK,), i32),
            pltpu.VMEM((ABLK,), i32),
            pltpu.VMEM((2 * grp + 32,), i32),
            pltpu.VMEM((2 * grp + 32,), i32),
            pltpu.VMEM((grp, F), f32),
            pltpu.VMEM((nacc + 8, F), f32),
            pltpu.VMEM((NC * 2 * NS * 16,), i32),
            pltpu.SemaphoreType.DMA,
        ],
    )


def _make_deg():
    grp = 128
    nj = 4
    nacc = nj * RNG

    def body(gsrc, gdst, cnts, out, sst, dst_, psrc, pldst, acc, cnt_v):
        c = lax.axis_index("c")
        s = lax.axis_index("s")
        nr = 62 + c
        half = c * B0
        pltpu.sync_copy(cnts, cnt_v)
        one16 = jnp.ones((16,), f32)

        def fire():
            for g16 in range(grp // 16):
                ldv = pldst[pl.ds(g16 * 16, 16)]
                for l in range(16):
                    plsc.addupdate(acc.at[ldv[l], pl.ds(0, 16)], one16)

        def zrow(zr, _):
            def zb(k, _):
                acc[zr, pl.ds(k * 16, 16)] = jnp.zeros((16,), f32)
                return 0

            return lax.fori_loop(0, 128 // 16, zb, 0)

        lax.fori_loop(0, nacc + 8, zrow, 0)
        ridx = [s + 16 * j for j in range(nj)]
        lo = [half + r * RNG for r in ridx]
        valid = [r < nr for r in ridx]

        def scan_shard(sh, pend):
            slot = c * 2 * NS + sh
            cv = cnt_v[pl.ds(slot * 16, 16)]
            cnt_sh = cv[0]
            nblk = (cnt_sh + (ABLK - 1)) // ABLK

            def blkb(b, pend):
                sbase = slot * CAP + b * ABLK
                pltpu.sync_copy(gdst.at[pl.ds(sbase, ABLK)], dst_)

                def inb(i, pend):
                    pos = b * ABLK + i * 16 + lax.broadcasted_iota(i32, (16,), 0)
                    d = dst_[pl.ds(i * 16, 16)]
                    m = jnp.zeros((16,), jnp.bool_)
                    ld = jnp.full((16,), nacc, i32)
                    for j in range(nj):
                        inj = (d >= lo[j]) & (d < lo[j] + RNG) & valid[j]
                        m = m | inj
                        ld = jnp.where(inj, d - lo[j] + j * RNG, ld)
                    m = m & (pos < cnt_sh)
                    run = plsc.cumsum(m.astype(i32))
                    p = pend + run - 1
                    plsc.store_scatter(pldst, [p], ld, mask=m)
                    pend = pend + run[15]

                    def do_fire(pend):
                        fire()
                        pldst[pl.ds(0, 16)] = pldst[pl.ds(grp, 16)]
                        return pend - grp

                    return lax.cond(pend >= grp, do_fire, lambda p: p, pend)

                return lax.fori_loop(0, ABLK // 16, inb, pend)

            return lax.fori_loop(0, nblk, blkb, pend)

        pend = lax.fori_loop(0, 2 * NS, scan_shard, jnp.int32(0))
        for j in range(grp // 16):
            pldst[pl.ds(pend + j * 16, 16)] = jnp.full((16,), nacc, i32)
        fire()
        for j in range(nj):
            @pl.when(valid[j])
            def _(j=j):
                pltpu.sync_copy(
                    acc.at[pl.ds(j * RNG, RNG)],
                    out.at[pl.ds(lo[j], RNG)],
                )

    return pl.kernel(
        body,
        out_type=jax.ShapeDtypeStruct((N, 128), f32),
        mesh=_sc_mesh(),
        compiler_params=pltpu.CompilerParams(needs_layout_passes=False),
        scratch_types=[
            pltpu.VMEM((16,), i32),
            pltpu.VMEM((ABLK,), i32),
            pltpu.VMEM((2 * grp + 32,), i32),
            pltpu.VMEM((2 * grp + 32,), i32),
            pltpu.VMEM((nacc + 8, 128), f32),
            pltpu.VMEM((NC * 2 * NS * 16,), i32),
        ],
    )


_R = 400


def _inv_sqrt(deg):
    return jnp.where(deg > 0, lax.rsqrt(jnp.maximum(deg, 1.0)), 0.0)


def _scale_body(x_ref, deg_ref, o_ref):
    inv = _inv_sqrt(deg_ref[:, :1])
    x = x_ref[...]
    o_ref[...] = jnp.concatenate(
        [x * inv, jnp.zeros((x.shape[0], 256 - x.shape[1]), f32)], axis=1
    )


def _tc_scale(x, deg_out):
    F = x.shape[1]
    return pl.pallas_call(
        _scale_body,
        grid=(N // _R,),
        in_specs=[
            pl.BlockSpec((_R, F), lambda i: (i, 0)),
            pl.BlockSpec((_R, 128), lambda i: (i, 0)),
        ],
        out_specs=pl.BlockSpec((_R, 256), lambda i: (i, 0)),
        out_shape=jax.ShapeDtypeStruct((N, 256), f32),
    )(x, deg_out)


def _mm_stats_body(agg_ref, deg_ref, w_ref, b_ref, z_ref, st_ref):
    i = pl.program_id(0)
    inv = _inv_sqrt(deg_ref[:, :1])
    z = jnp.dot(agg_ref[...] * inv, w_ref[...], preferred_element_type=f32)
    z = z + b_ref[...]
    z_ref[...] = z
    s0 = jnp.sum(z, axis=0, keepdims=True)
    s1 = jnp.sum(z * z, axis=0, keepdims=True)
    part = jnp.concatenate([s0, s1, jnp.zeros((6, z.shape[1]), f32)], axis=0)

    @pl.when(i == 0)
    def _():
        st_ref[...] = part

    @pl.when(i > 0)
    def _():
        st_ref[...] += part


def _tc_mm_stats(agg, deg_in, W, b):
    Fi, Fo = W.shape
    return pl.pallas_call(
        _mm_stats_body,
        grid=(N // _R,),
        in_specs=[
            pl.BlockSpec((_R, Fi), lambda i: (i, 0)),
            pl.BlockSpec((_R, 128), lambda i: (i, 0)),
            pl.BlockSpec((Fi, Fo), lambda i: (0, 0)),
            pl.BlockSpec((1, Fo), lambda i: (0, 0)),
        ],
        out_specs=[
            pl.BlockSpec((_R, Fo), lambda i: (i, 0)),
            pl.BlockSpec((8, Fo), lambda i: (0, 0)),
        ],
        out_shape=[
            jax.ShapeDtypeStruct((N, Fo), f32),
            jax.ShapeDtypeStruct((8, Fo), f32),
        ],
    )(agg, deg_in, W, b.reshape(1, Fo))


def _bn_relu_body(z_ref, st_ref, deg_ref, o_ref):
    mean = st_ref[0:1, :] * (1.0 / N)
    ex2 = st_ref[1:2, :] * (1.0 / N)
    var = ex2 - mean * mean
    rstd = lax.rsqrt(var + 1e-5)
    h = jnp.maximum((z_ref[...] - mean) * rstd, 0.0)
    inv = _inv_sqrt(deg_ref[:, :1])
    o_ref[...] = h * inv


def _tc_bn_relu(z, stats, deg_out):
    Fo = z.shape[1]
    return pl.pallas_call(
        _bn_relu_body,
        grid=(N // _R,),
        in_specs=[
            pl.BlockSpec((_R, Fo), lambda i: (i, 0)),
            pl.BlockSpec((8, Fo), lambda i: (0, 0)),
            pl.BlockSpec((_R, 128), lambda i: (i, 0)),
        ],
        out_specs=pl.BlockSpec((_R, Fo), lambda i: (i, 0)),
        out_shape=jax.ShapeDtypeStruct((N, Fo), f32),
    )(z, stats, deg_out)


def _mm_body(h_ref, w_ref, o_ref):
    o_ref[...] = jnp.dot(h_ref[...], w_ref[...], preferred_element_type=f32)


def _tc_mm(h, W):
    Fi, Fo = W.shape
    return pl.pallas_call(
        _mm_body,
        grid=(N // _R,),
        in_specs=[
            pl.BlockSpec((_R, Fi), lambda i: (i, 0)),
            pl.BlockSpec((Fi, Fo), lambda i: (0, 0)),
        ],
        out_specs=pl.BlockSpec((_R, Fo), lambda i: (i, 0)),
        out_shape=jax.ShapeDtypeStruct((N, Fo), f32),
    )(h, W)


def _softmax_body(agg_ref, deg_ref, b_ref, o_ref):
    inv = _inv_sqrt(deg_ref[:, :1])
    t = agg_ref[...] * inv + b_ref[...]
    logits = t[:, :2]
    m = jnp.max(logits, axis=1, keepdims=True)
    e = jnp.exp(logits - m)
    o_ref[...] = e / jnp.sum(e, axis=1, keepdims=True)


def _tc_softmax(aggy, deg_in, b6p):
    return pl.pallas_call(
        _softmax_body,
        grid=(N // _R,),
        in_specs=[
            pl.BlockSpec((_R, 128), lambda i: (i, 0)),
            pl.BlockSpec((_R, 128), lambda i: (i, 0)),
            pl.BlockSpec((1, 128), lambda i: (0, 0)),
        ],
        out_specs=pl.BlockSpec((_R, 2), lambda i: (i, 0)),
        out_shape=jax.ShapeDtypeStruct((N, 2), f32),
    )(aggy, deg_in, b6p)


def kernel(x, edge_index, W0, b0, W1, b1, W2, b2, W3, b3, W4, b4, W5, b5, W6, b6):
    ei = edge_index.astype(i32)
    src_arr = ei[0]
    dst_arr = ei[1]
    part = _make_partition()
    part_dst = part(dst_arr, src_arr)
    part_src = part(src_arr, dst_arr)

    agg256 = _make_agg(256)
    agg512 = _make_agg(512)
    deg = _make_deg()
    deg_in = deg(*part_dst)
    deg_out = deg(*part_src)

    W0p = jnp.pad(W0, ((0, 128), (0, 0)))
    Ws = [W0p, W1, W2, W3, W4, W5]
    bs = [b0, b1, b2, b3, b4, b5]
    h = _tc_scale(x, deg_out)
    for k in range(6):
        agg = (agg256 if h.shape[1] == 256 else agg512)(h, *part_dst)
        z, st = _tc_mm_stats(agg, deg_in, Ws[k], bs[k])
        h = _tc_bn_relu(z, st, deg_out)
    W6p = jnp.pad(W6, ((0, 0), (0, 254)))
    y = _tc_mm(h, W6p)
    aggy = agg256(y, *part_dst)
    b6p = jnp.pad(b6, (0, 254)).reshape(1, 256)
    return _tc_softmax(aggy, deg_in, b6p)


# speedup vs baseline: 1.0284x; 1.0284x over previous
import jax
import jax.numpy as jnp
from jax import lax
from jax.experimental import pallas as pl
from jax.experimental.pallas import tpu as pltpu
from jax.experimental.pallas import tpu_sc as plsc

N = 10000
E = 640000
NC = 2
NS = 16
B0 = 4960
RNG = 80
EPT = E // NS
BLK = 1600
ABLK = 1600
GRP = 128
CAP = 43264

i32 = jnp.int32
f32 = jnp.float32


def _sc_mesh():
    return plsc.VectorSubcoreMesh(core_axis_name="c", subcore_axis_name="s")


def _make_partition():
    def body(key_arr, oth_arr, gsrc, gdst, cnts, key_st, oth_st, csrc, cdst, cnt_st):
        c = lax.axis_index("c")
        s = lax.axis_index("s")
        base_e = s * EPT
        half_hi = B0 + c * (2 * N)
        for bkt in range(2):
            lo = c * B0 + bkt * (32 * RNG)
            hi = jnp.minimum(lo + 32 * RNG, half_hi)

            def blk_body(b, cnt):
                off = base_e + b * BLK
                pltpu.sync_copy(key_arr.at[pl.ds(off, BLK)], key_st)
                pltpu.sync_copy(oth_arr.at[pl.ds(off, BLK)], oth_st)

                def in_body(i, cnt):
                    d = key_st[pl.ds(i * 16, 16)]
                    o = oth_st[pl.ds(i * 16, 16)]
                    m = (d >= lo) & (d < hi)
                    run = plsc.cumsum(m.astype(i32))
                    pos = cnt + run - 1
                    plsc.store_scatter(csrc, [pos], o, mask=m)
                    plsc.store_scatter(cdst, [pos], d, mask=m)
                    return cnt + run[15]

                return lax.fori_loop(0, BLK // 16, in_body, cnt)

            cnt = lax.fori_loop(0, EPT // BLK, blk_body, jnp.int32(0))
            slot = (c * 2 + bkt) * NS + s
            fbase = slot * CAP
            pltpu.sync_copy(csrc.at[pl.ds(0, CAP)], gsrc.at[pl.ds(fbase, CAP)])
            pltpu.sync_copy(cdst.at[pl.ds(0, CAP)], gdst.at[pl.ds(fbase, CAP)])
            cnt_st[...] = jnp.full((16,), cnt, i32)
            pltpu.sync_copy(cnt_st, cnts.at[pl.ds(slot * 16, 16)])

    return pl.kernel(
        body,
        out_type=[
            jax.ShapeDtypeStruct((NC * 2 * NS * CAP,), i32),
            jax.ShapeDtypeStruct((NC * 2 * NS * CAP,), i32),
            jax.ShapeDtypeStruct((NC * 2 * NS * 16,), i32),
        ],
        mesh=_sc_mesh(),
        compiler_params=pltpu.CompilerParams(needs_layout_passes=False),
        scratch_types=[
            pltpu.VMEM((BLK,), i32),
            pltpu.VMEM((BLK,), i32),
            pltpu.VMEM((CAP + GRP,), i32),
            pltpu.VMEM((CAP + GRP,), i32),
            pltpu.VMEM((16,), i32),
        ],
    )


def _make_agg(F: int):
    grp = 128 if F <= 256 else 64
    nj = 4 if F <= 256 else 2
    nscan = 1 if F <= 256 else 2
    nacc = nj * RNG

    def body(h, gsrc, gdst, cnts, out, sst, dst_, psrc, pldst, rows, acc, cnt_v, sem, sem2):
        c = lax.axis_index("c")
        s = lax.axis_index("s")
        nr = 62 + c
        half = c * B0
        pltpu.sync_copy(cnts, cnt_v)

        def fire():
            hg = grp // 2
            cp1 = pltpu.async_copy(
                h.at[psrc.at[pl.ds(0, hg)]], rows.at[pl.ds(0, hg)], sem)
            cp2 = pltpu.async_copy(
                h.at[psrc.at[pl.ds(hg, hg)]], rows.at[pl.ds(hg, hg)], sem2)

            def acc_half(base):
                for g16 in range(hg // 16):
                    ldv = pldst[pl.ds(base + g16 * 16, 16)]
                    ds = [ldv[l] for l in range(16)]

                    def addf(k, _, g16=g16, ds=ds, base=base):
                        for l in range(16):
                            v = rows[base + g16 * 16 + l, pl.ds(k * 16, 16)]
                            plsc.addupdate(acc.at[ds[l], pl.ds(k * 16, 16)], v)
                        return 0

                    lax.fori_loop(0, F // 16, addf, 0)

            cp1.wait()
            acc_half(0)
            cp2.wait()
            acc_half(hg)

        for q in range(nscan):
            def zrow(zr, _):
                def zb(k, _):
                    acc[zr, pl.ds(k * 16, 16)] = jnp.zeros((16,), f32)
                    return 0

                return lax.fori_loop(0, F // 16, zb, 0)

            lax.fori_loop(0, nacc + 8, zrow, 0)
            ridx = [s + 16 * (nj * q + j) for j in range(nj)]
            lo = [half + r * RNG for r in ridx]
            valid = [r < nr for r in ridx]
            slot0 = (c * 2 + (q if nscan == 2 else 0)) * NS
            nslot = NS if nscan == 2 else 2 * NS

            def scan_shard(sh, pend):
                slot = slot0 + sh
                cv = cnt_v[pl.ds(slot * 16, 16)]
                cnt_sh = cv[0]
                nblk = (cnt_sh + (ABLK - 1)) // ABLK

                def blkb(b, pend):
                    sbase = slot * CAP + b * ABLK
                    pltpu.sync_copy(gsrc.at[pl.ds(sbase, ABLK)], sst)
                    pltpu.sync_copy(gdst.at[pl.ds(sbase, ABLK)], dst_)

                    def inb(i, pend):
                        pos = b * ABLK + i * 16 + lax.broadcasted_iota(i32, (16,), 0)
                        d = dst_[pl.ds(i * 16, 16)]
                        o = sst[pl.ds(i * 16, 16)]
                        m = jnp.zeros((16,), jnp.bool_)
                        ld = jnp.full((16,), nacc, i32)
                        for j in range(nj):
                            inj = (d >= lo[j]) & (d < lo[j] + RNG) & valid[j]
                            m = m | inj
                            ld = jnp.where(inj, d - lo[j] + j * RNG, ld)
                        m = m & (pos < cnt_sh)
                        run = plsc.cumsum(m.astype(i32))
                        p = pend + run - 1
                        plsc.store_scatter(psrc, [p], o, mask=m)
                        plsc.store_scatter(pldst, [p], ld, mask=m)
                        pend = pend + run[15]

                        def do_fire(pend):
                            fire()
                            psrc[pl.ds(0, 16)] = psrc[pl.ds(grp, 16)]
                            pldst[pl.ds(0, 16)] = pldst[pl.ds(grp, 16)]
                            return pend - grp

                        return lax.cond(pend >= grp, do_fire, lambda p: p, pend)

                    return lax.fori_loop(0, ABLK // 16, inb, pend)

                return lax.fori_loop(0, nblk, blkb, pend)

            pend = lax.fori_loop(0, nslot, scan_shard, jnp.int32(0))
            for j in range(grp // 16):
                psrc[pl.ds(pend + j * 16, 16)] = jnp.zeros((16,), i32)
                pldst[pl.ds(pend + j * 16, 16)] = jnp.full((16,), nacc, i32)
            fire()
            for j in range(nj):
                @pl.when(valid[j])
                def _(j=j):
                    pltpu.sync_copy(
                        acc.at[pl.ds(j * RNG, RNG)],
                        out.at[pl.ds(lo[j], RNG)],
                    )

    return pl.kernel(
        body,
        out_type=jax.ShapeDtypeStruct((N, F), f32),
        mesh=_sc_mesh(),
        compiler_params=pltpu.CompilerParams(needs_layout_passes=False),
        scratch_types=[
            pltpu.VMEM((ABLK,), i32),
            pltpu.VMEM((ABLK,), i32),
            pltpu.VMEM((2 * grp + 32,), i32),
            pltpu.VMEM((2 * grp + 32,), i32),
            pltpu.VMEM((grp, F), f32),
            pltpu.VMEM((nacc + 8, F), f32),
            pltpu.VMEM((NC * 2 * NS * 16,), i32),
            pltpu.SemaphoreType.DMA,
            pltpu.SemaphoreType.DMA,
        ],
    )


def _make_deg():
    grp = 128
    nj = 4
    nacc = nj * RNG

    def body(gsrc, gdst, cnts, out, sst, dst_, psrc, pldst, acc, cnt_v):
        c = lax.axis_index("c")
        s = lax.axis_index("s")
        nr = 62 + c
        half = c * B0
        pltpu.sync_copy(cnts, cnt_v)
        one16 = jnp.ones((16,), f32)

        def fire():
            for g16 in range(grp // 16):
                ldv = pldst[pl.ds(g16 * 16, 16)]
                for l in range(16):
                    plsc.addupdate(acc.at[ldv[l], pl.ds(0, 16)], one16)

        def zrow(zr, _):
            def zb(k, _):
                acc[zr, pl.ds(k * 16, 16)] = jnp.zeros((16,), f32)
                return 0

            return lax.fori_loop(0, 128 // 16, zb, 0)

        lax.fori_loop(0, nacc + 8, zrow, 0)
        ridx = [s + 16 * j for j in range(nj)]
        lo = [half + r * RNG for r in ridx]
        valid = [r < nr for r in ridx]

        def scan_shard(sh, pend):
            slot = c * 2 * NS + sh
            cv = cnt_v[pl.ds(slot * 16, 16)]
            cnt_sh = cv[0]
            nblk = (cnt_sh + (ABLK - 1)) // ABLK

            def blkb(b, pend):
                sbase = slot * CAP + b * ABLK
                pltpu.sync_copy(gdst.at[pl.ds(sbase, ABLK)], dst_)

                def inb(i, pend):
                    pos = b * ABLK + i * 16 + lax.broadcasted_iota(i32, (16,), 0)
                    d = dst_[pl.ds(i * 16, 16)]
                    m = jnp.zeros((16,), jnp.bool_)
                    ld = jnp.full((16,), nacc, i32)
                    for j in range(nj):
                        inj = (d >= lo[j]) & (d < lo[j] + RNG) & valid[j]
                        m = m | inj
                        ld = jnp.where(inj, d - lo[j] + j * RNG, ld)
                    m = m & (pos < cnt_sh)
                    run = plsc.cumsum(m.astype(i32))
                    p = pend + run - 1
                    plsc.store_scatter(pldst, [p], ld, mask=m)
                    pend = pend + run[15]

                    def do_fire(pend):
                        fire()
                        pldst[pl.ds(0, 16)] = pldst[pl.ds(grp, 16)]
                        return pend - grp

                    return lax.cond(pend >= grp, do_fire, lambda p: p, pend)

                return lax.fori_loop(0, ABLK // 16, inb, pend)

            return lax.fori_loop(0, nblk, blkb, pend)

        pend = lax.fori_loop(0, 2 * NS, scan_shard, jnp.int32(0))
        for j in range(grp // 16):
            pldst[pl.ds(pend + j * 16, 16)] = jnp.full((16,), nacc, i32)
        fire()
        for j in range(nj):
            @pl.when(valid[j])
            def _(j=j):
                pltpu.sync_copy(
                    acc.at[pl.ds(j * RNG, RNG)],
                    out.at[pl.ds(lo[j], RNG)],
                )

    return pl.kernel(
        body,
        out_type=jax.ShapeDtypeStruct((N, 128), f32),
        mesh=_sc_mesh(),
        compiler_params=pltpu.CompilerParams(needs_layout_passes=False),
        scratch_types=[
            pltpu.VMEM((16,), i32),
            pltpu.VMEM((ABLK,), i32),
            pltpu.VMEM((2 * grp + 32,), i32),
            pltpu.VMEM((2 * grp + 32,), i32),
            pltpu.VMEM((nacc + 8, 128), f32),
            pltpu.VMEM((NC * 2 * NS * 16,), i32),
        ],
    )


_R = 400


def _inv_sqrt(deg):
    return jnp.where(deg > 0, lax.rsqrt(jnp.maximum(deg, 1.0)), 0.0)


def _scale_body(x_ref, deg_ref, o_ref):
    inv = _inv_sqrt(deg_ref[:, :1])
    x = x_ref[...]
    o_ref[...] = jnp.concatenate(
        [x * inv, jnp.zeros((x.shape[0], 256 - x.shape[1]), f32)], axis=1
    )


def _tc_scale(x, deg_out):
    F = x.shape[1]
    return pl.pallas_call(
        _scale_body,
        grid=(N // _R,),
        in_specs=[
            pl.BlockSpec((_R, F), lambda i: (i, 0)),
            pl.BlockSpec((_R, 128), lambda i: (i, 0)),
        ],
        out_specs=pl.BlockSpec((_R, 256), lambda i: (i, 0)),
        out_shape=jax.ShapeDtypeStruct((N, 256), f32),
    )(x, deg_out)


def _mm_stats_body(agg_ref, deg_ref, w_ref, b_ref, z_ref, st_ref):
    i = pl.program_id(0)
    inv = _inv_sqrt(deg_ref[:, :1])
    z = jnp.dot(agg_ref[...] * inv, w_ref[...], preferred_element_type=f32)
    z = z + b_ref[...]
    z_ref[...] = z
    s0 = jnp.sum(z, axis=0, keepdims=True)
    s1 = jnp.sum(z * z, axis=0, keepdims=True)
    part = jnp.concatenate([s0, s1, jnp.zeros((6, z.shape[1]), f32)], axis=0)

    @pl.when(i == 0)
    def _():
        st_ref[...] = part

    @pl.when(i > 0)
    def _():
        st_ref[...] += part


def _tc_mm_stats(agg, deg_in, W, b):
    Fi, Fo = W.shape
    return pl.pallas_call(
        _mm_stats_body,
        grid=(N // _R,),
        in_specs=[
            pl.BlockSpec((_R, Fi), lambda i: (i, 0)),
            pl.BlockSpec((_R, 128), lambda i: (i, 0)),
            pl.BlockSpec((Fi, Fo), lambda i: (0, 0)),
            pl.BlockSpec((1, Fo), lambda i: (0, 0)),
        ],
        out_specs=[
            pl.BlockSpec((_R, Fo), lambda i: (i, 0)),
            pl.BlockSpec((8, Fo), lambda i: (0, 0)),
        ],
        out_shape=[
            jax.ShapeDtypeStruct((N, Fo), f32),
            jax.ShapeDtypeStruct((8, Fo), f32),
        ],
    )(agg, deg_in, W, b.reshape(1, Fo))


def _bn_relu_body(z_ref, st_ref, deg_ref, o_ref):
    mean = st_ref[0:1, :] * (1.0 / N)
    ex2 = st_ref[1:2, :] * (1.0 / N)
    var = ex2 - mean * mean
    rstd = lax.rsqrt(var + 1e-5)
    h = jnp.maximum((z_ref[...] - mean) * rstd, 0.0)
    inv = _inv_sqrt(deg_ref[:, :1])
    o_ref[...] = h * inv


def _tc_bn_relu(z, stats, deg_out):
    Fo = z.shape[1]
    return pl.pallas_call(
        _bn_relu_body,
        grid=(N // _R,),
        in_specs=[
            pl.BlockSpec((_R, Fo), lambda i: (i, 0)),
            pl.BlockSpec((8, Fo), lambda i: (0, 0)),
            pl.BlockSpec((_R, 128), lambda i: (i, 0)),
        ],
        out_specs=pl.BlockSpec((_R, Fo), lambda i: (i, 0)),
        out_shape=jax.ShapeDtypeStruct((N, Fo), f32),
    )(z, stats, deg_out)


def _mm_body(h_ref, w_ref, o_ref):
    o_ref[...] = jnp.dot(h_ref[...], w_ref[...], preferred_element_type=f32)


def _tc_mm(h, W):
    Fi, Fo = W.shape
    return pl.pallas_call(
        _mm_body,
        grid=(N // _R,),
        in_specs=[
            pl.BlockSpec((_R, Fi), lambda i: (i, 0)),
            pl.BlockSpec((Fi, Fo), lambda i: (0, 0)),
        ],
        out_specs=pl.BlockSpec((_R, Fo), lambda i: (i, 0)),
        out_shape=jax.ShapeDtypeStruct((N, Fo), f32),
    )(h, W)


def _softmax_body(agg_ref, deg_ref, b_ref, o_ref):
    inv = _inv_sqrt(deg_ref[:, :1])
    t = agg_ref[...] * inv + b_ref[...]
    logits = t[:, :2]
    m = jnp.max(logits, axis=1, keepdims=True)
    e = jnp.exp(logits - m)
    o_ref[...] = e / jnp.sum(e, axis=1, keepdims=True)


def _tc_softmax(aggy, deg_in, b6p):
    return pl.pallas_call(
        _softmax_body,
        grid=(N // _R,),
        in_specs=[
            pl.BlockSpec((_R, 128), lambda i: (i, 0)),
            pl.BlockSpec((_R, 128), lambda i: (i, 0)),
            pl.BlockSpec((1, 128), lambda i: (0, 0)),
        ],
        out_specs=pl.BlockSpec((_R, 2), lambda i: (i, 0)),
        out_shape=jax.ShapeDtypeStruct((N, 2), f32),
    )(aggy, deg_in, b6p)


def kernel(x, edge_index, W0, b0, W1, b1, W2, b2, W3, b3, W4, b4, W5, b5, W6, b6):
    ei = edge_index.astype(i32)
    src_arr = ei[0]
    dst_arr = ei[1]
    part = _make_partition()
    part_dst = part(dst_arr, src_arr)
    part_src = part(src_arr, dst_arr)

    agg256 = _make_agg(256)
    agg512 = _make_agg(512)
    deg = _make_deg()
    deg_in = deg(*part_dst)
    deg_out = deg(*part_src)

    W0p = jnp.pad(W0, ((0, 128), (0, 0)))
    Ws = [W0p, W1, W2, W3, W4, W5]
    bs = [b0, b1, b2, b3, b4, b5]
    h = _tc_scale(x, deg_out)
    for k in range(6):
        agg = (agg256 if h.shape[1] == 256 else agg512)(h, *part_dst)
        z, st = _tc_mm_stats(agg, deg_in, Ws[k], bs[k])
        h = _tc_bn_relu(z, st, deg_out)
    W6p = jnp.pad(W6, ((0, 0), (0, 254)))
    y = _tc_mm(h, W6p)
    aggy = agg256(y, *part_dst)
    b6p = jnp.pad(b6, (0, 254)).reshape(1, 256)
    return _tc_softmax(aggy, deg_in, b6p)
